# Initial kernel scaffold; baseline (speedup 1.0000x reference)
import jax
import jax.numpy as jnp
from jax.experimental import pallas as pl


def kernel(node_feats, coords, cond_feats, V1, g1, b1, V2, g2, b2):
    raise NotImplementedError("write your pallas kernel here")


# speedup vs baseline: 1.0442x; 1.0442x over previous
import functools

import jax
import jax.numpy as jnp
from jax import lax
from jax.experimental import pallas as pl
from jax.experimental.pallas import tpu as pltpu

B, N = 16, 1024
D_FEAT, D_COORD, COND = 253, 3, 512
DX = D_FEAT + D_COORD
HID = 512
K = 32

_P = lax.Precision.DEFAULT


def _dot_t(a, b, precision=_P):
    return lax.dot_general(a, b, (((1,), (1,)), ((), ())),
                           preferred_element_type=jnp.float32,
                           precision=precision)


def _fused_body(x_ref, cond_ref, W1_ref, b1_ref, W2_ref, b2_ref, out_ref):
    b = pl.program_id(0)
    x = x_ref[0]
    cond = cond_ref[pl.ds(b, 1), :]
    q = jnp.concatenate(
        [x, jnp.broadcast_to(cond, (N, COND))], axis=1)
    h = jnp.maximum(_dot_t(q, W1_ref[...]) + b1_ref[...][None, :], 0.0)
    f = jnp.maximum(_dot_t(h, W2_ref[...]) + b2_ref[...][None, :], 0.0)
    out_ref[0] = _dot_t(f, f)


def _logits(x, cond_feats, W1, b1, W2, b2):
    return pl.pallas_call(
        _fused_body,
        grid=(B,),
        in_specs=[
            pl.BlockSpec((1, N, DX), lambda b: (b, 0, 0)),
            pl.BlockSpec((B, COND), lambda b: (0, 0)),
            pl.BlockSpec((HID, DX + COND), lambda b: (0, 0)),
            pl.BlockSpec((HID,), lambda b: (0,)),
            pl.BlockSpec((HID, HID), lambda b: (0, 0)),
            pl.BlockSpec((HID,), lambda b: (0,)),
        ],
        out_specs=pl.BlockSpec((1, N, N), lambda b: (b, 0, 0)),
        out_shape=jax.ShapeDtypeStruct((B, N, N), jnp.float32),
    )(x, cond_feats, W1, b1, W2, b2)


def _weight_norm_setup(V, g):
    return g[:, None] * V / jnp.linalg.norm(V, axis=1, keepdims=True)


def kernel(node_feats, coords, cond_feats, V1, g1, b1, V2, g2, b2):
    x = jnp.concatenate([node_feats, coords], axis=-1)
    W1 = _weight_norm_setup(V1, g1)
    W2 = _weight_norm_setup(V2, g2)
    logits = _logits(x, cond_feats, W1, b1, W2, b2)
    topk_logits, topk_idx = lax.top_k(logits, K)
    edge_weights = jax.nn.softmax(topk_logits, axis=-1)
    return edge_weights, topk_idx


# speedup vs baseline: 3.8934x; 3.7286x over previous
import functools

import jax
import jax.numpy as jnp
from jax import lax
from jax.experimental import pallas as pl
from jax.experimental.pallas import tpu as pltpu
from jax.experimental.pallas import tpu_sc as plsc

B, N = 16, 1024
D_FEAT, D_COORD, COND = 253, 3, 512
DX = D_FEAT + D_COORD
HID = 512
K = 32

_P = lax.Precision.DEFAULT


def _dot_t(a, b, precision=_P):
    return lax.dot_general(a, b, (((1,), (1,)), ((), ())),
                           preferred_element_type=jnp.float32,
                           precision=precision)


def _fused_body(x_ref, cond_ref, W1_ref, b1_ref, W2_ref, b2_ref, out_ref):
    b = pl.program_id(0)
    x = x_ref[0]
    cond = cond_ref[pl.ds(b, 1), :]
    q = jnp.concatenate(
        [x, jnp.broadcast_to(cond, (N, COND))], axis=1)
    h = jnp.maximum(_dot_t(q, W1_ref[...]) + b1_ref[...][None, :], 0.0)
    f = jnp.maximum(_dot_t(h, W2_ref[...]) + b2_ref[...][None, :], 0.0)
    out_ref[0] = _dot_t(f, f)


def _logits(x, cond_feats, W1, b1, W2, b2):
    return pl.pallas_call(
        _fused_body,
        grid=(B,),
        in_specs=[
            pl.BlockSpec((1, N, DX), lambda b: (b, 0, 0)),
            pl.BlockSpec((B, COND), lambda b: (0, 0)),
            pl.BlockSpec((HID, DX + COND), lambda b: (0, 0)),
            pl.BlockSpec((HID,), lambda b: (0,)),
            pl.BlockSpec((HID, HID), lambda b: (0, 0)),
            pl.BlockSpec((HID,), lambda b: (0,)),
        ],
        out_specs=pl.BlockSpec((1, N, N), lambda b: (b, 0, 0)),
        out_shape=jax.ShapeDtypeStruct((B, N, N), jnp.float32),
    )(x, cond_feats, W1, b1, W2, b2)


L = 16
NW = 32
ROWS = B * N
RPW = ROWS // NW
R = 8
NBLK = RPW // R
NCH = N // L
CAP = N + L
_NINF = float("-inf")


def _merge_desc(ak, av, bk, bv):
    rbk = lax.rev(bk, (0,))
    rbv = lax.rev(bv, (0,))
    m = ak >= rbk
    tk = jnp.where(m, ak, rbk)
    tv = jnp.where(m, av, rbv)
    uk = jnp.where(m, rbk, ak)
    uv = jnp.where(m, rbv, av)
    tk, tv = plsc.sort_key_val(tk, tv, descending=True)
    uk, uv = plsc.sort_key_val(uk, uv, descending=True)
    return tk, tv, uk, uv


def _sorted_chunk(cand_v, cand_i, j):
    ck, cv = plsc.sort_key_val(cand_v[pl.ds(j * L, L)],
                               cand_i[pl.ds(j * L, L)], descending=True)
    return ck, cv


def _row_topk(buf, cand_v, cand_i, ow, oi, tv, ti, i):
    def p1(j, carry):
        m1, m2 = carry
        c = buf[i, pl.ds(j * L, L)]
        m2 = jnp.maximum(m2, jnp.minimum(m1, c))
        m1 = jnp.maximum(m1, c)
        return m1, m2
    _, m2 = lax.fori_loop(0, NCH, p1,
                          (jnp.full((L,), _NINF, jnp.float32), jnp.full((L,), _NINF, jnp.float32)))
    t0 = jnp.min(m2)

    iota = lax.iota(jnp.int32, L)

    def p2(j, cnt):
        c = buf[i, pl.ds(j * L, L)]
        msk = c >= t0
        plsc.store_compressed(cand_v.at[pl.ds(cnt, L)], c, mask=msk)
        plsc.store_compressed(cand_i.at[pl.ds(cnt, L)], iota + j * L, mask=msk)
        return cnt + jnp.sum(msk.astype(jnp.int32))
    cnt = lax.fori_loop(0, NCH, p2, jnp.int32(0))
    cand_v[pl.ds(cnt, L)] = jnp.full((L,), _NINF, jnp.float32)

    k0, v0 = _sorted_chunk(cand_v, cand_i, 0)
    k1, v1 = _sorted_chunk(cand_v, cand_i, 1)
    hk, hv, lk, lv = _merge_desc(k0, v0, k1, v1)
    nch = (cnt + (L - 1)) // L

    def cond(st):
        return st[0] < nch

    def body(st):
        j, hk, hv, lk, lv = st
        ck, cv = _sorted_chunk(cand_v, cand_i, j)
        yk, yv, _, _ = _merge_desc(lk, lv, ck, cv)
        hk, hv, lk, lv = _merge_desc(hk, hv, yk, yv)
        return j + 1, hk, hv, lk, lv

    _, hk, hv, lk, lv = lax.while_loop(cond, body,
                                       (jnp.int32(2), hk, hv, lk, lv))

    tv[pl.ds(0, L)] = jnp.full((L,), float("inf"), jnp.float32)
    tv[pl.ds(L, L)] = hk
    tv[pl.ds(2 * L, L)] = lk
    tv[pl.ds(3 * L, L)] = jnp.full((L,), _NINF, jnp.float32)
    vp_h = tv[pl.ds(L - 1, L)]
    vn_h = tv[pl.ds(L + 1, L)]
    vp_l = tv[pl.ds(2 * L - 1, L)]
    vn_l = tv[pl.ds(2 * L + 1, L)]
    eqp_h = hk == vp_h
    eqn_h = hk == vn_h
    eqp_l = lk == vp_l
    eqn_l = lk == vn_l
    par = (lax.iota(jnp.int32, L) % 2) == 0
    npar = jnp.logical_not(par)
    ti[pl.ds(0, L)] = lax.iota(jnp.int32, L)
    ti[pl.ds(3 * L, L)] = lax.iota(jnp.int32, L)
    for down in (par, npar, par):
        up = jnp.logical_not(down)
        ti[pl.ds(L, L)] = hv
        ti[pl.ds(2 * L, L)] = lv
        ip_h = ti[pl.ds(L - 1, L)]
        in_h = ti[pl.ds(L + 1, L)]
        ip_l = ti[pl.ds(2 * L - 1, L)]
        in_l = ti[pl.ds(2 * L + 1, L)]
        hv = jnp.where(down & eqn_h & (hv > in_h), in_h,
                       jnp.where(up & eqp_h & (ip_h > hv), ip_h, hv))
        lv = jnp.where(down & eqn_l & (lv > in_l), in_l,
                       jnp.where(up & eqp_l & (ip_l > lv), ip_l, lv))

    mx = jnp.max(hk)
    eh = jnp.exp(hk - mx)
    el = jnp.exp(lk - mx)
    s = jnp.sum(eh) + jnp.sum(el)
    ow[i, pl.ds(0, L)] = eh / s
    ow[i, pl.ds(L, L)] = el / s
    oi[i, pl.ds(0, L)] = hv
    oi[i, pl.ds(L, L)] = lv


def _topk_sc(logits):
    lr = logits.reshape(ROWS, N)
    mesh = plsc.VectorSubcoreMesh(core_axis_name="c", subcore_axis_name="s")

    @functools.partial(
        pl.kernel, mesh=mesh,
        compiler_params=pltpu.CompilerParams(needs_layout_passes=False),
        out_type=[jax.ShapeDtypeStruct((ROWS, K), jnp.float32),
                  jax.ShapeDtypeStruct((ROWS, K), jnp.int32)],
        scratch_types=[
            pltpu.VMEM((R, N), jnp.float32),
            pltpu.VMEM((R, N), jnp.float32),
            pltpu.VMEM((CAP,), jnp.float32),
            pltpu.VMEM((CAP,), jnp.int32),
            pltpu.VMEM((R, K), jnp.float32),
            pltpu.VMEM((R, K), jnp.int32),
            pltpu.VMEM((4 * L,), jnp.float32),
            pltpu.VMEM((4 * L,), jnp.int32),
            pltpu.SemaphoreType.DMA,
            pltpu.SemaphoreType.DMA,
        ],
    )
    def topk_kernel(lg, ow_hbm, oi_hbm, bufa, bufb, cand_v, cand_i,
                    ow, oi, tv, ti, sema, semb):
        wid = lax.axis_index("c") * 16 + lax.axis_index("s")
        base = wid * RPW
        pltpu.async_copy(lg.at[pl.ds(base, R)], bufa, sema)

        def pair(g, carry):
            for t in range(2):
                blk = g * 2 + t
                buf, sem = (bufa, sema) if t == 0 else (bufb, semb)
                obuf, osem = (bufb, semb) if t == 0 else (bufa, sema)
                rbase = base + blk * R
                pltpu.make_async_copy(lg.at[pl.ds(rbase, R)], buf, sem).wait()
                if t == 0:
                    pltpu.async_copy(lg.at[pl.ds(rbase + R, R)], obuf, osem)
                else:
                    @pl.when(blk + 1 < NBLK)
                    def _issue():
                        pltpu.async_copy(lg.at[pl.ds(rbase + R, R)],
                                         obuf, osem)

                def rows(i, c2):
                    _row_topk(buf, cand_v, cand_i, ow, oi, tv, ti, i)
                    return c2
                lax.fori_loop(0, R, rows, 0)
                pltpu.sync_copy(ow, ow_hbm.at[pl.ds(rbase, R)])
                pltpu.sync_copy(oi, oi_hbm.at[pl.ds(rbase, R)])
            return carry

        lax.fori_loop(0, NBLK // 2, pair, 0)

    ow, oi = topk_kernel(lr)
    return ow.reshape(B, N, K), oi.reshape(B, N, K)


def _weight_norm_setup(V, g):
    return g[:, None] * V / jnp.linalg.norm(V, axis=1, keepdims=True)


def kernel(node_feats, coords, cond_feats, V1, g1, b1, V2, g2, b2):
    x = jnp.concatenate([node_feats, coords], axis=-1)
    W1 = _weight_norm_setup(V1, g1)
    W2 = _weight_norm_setup(V2, g2)
    logits = _logits(x, cond_feats, W1, b1, W2, b2)
    edge_weights, topk_idx = _topk_sc(logits)
    return edge_weights, topk_idx


# speedup vs baseline: 4.0147x; 1.0312x over previous
import functools

import jax
import jax.numpy as jnp
from jax import lax
from jax.experimental import pallas as pl
from jax.experimental.pallas import tpu as pltpu
from jax.experimental.pallas import tpu_sc as plsc

B, N = 16, 1024
D_FEAT, D_COORD, COND = 253, 3, 512
DX = D_FEAT + D_COORD
HID = 512
K = 32

_P = lax.Precision.DEFAULT


def _dot_t(a, b, precision=_P):
    return lax.dot_general(a, b, (((1,), (1,)), ((), ())),
                           preferred_element_type=jnp.float32,
                           precision=precision)


def _fused_body(x_ref, cond_ref, W1_ref, b1_ref, W2_ref, b2_ref, out_ref):
    b = pl.program_id(0)
    x = x_ref[0]
    cond = cond_ref[pl.ds(b, 1), :]
    q = jnp.concatenate(
        [x, jnp.broadcast_to(cond, (N, COND))], axis=1)
    h = jnp.maximum(_dot_t(q, W1_ref[...]) + b1_ref[...][None, :], 0.0)
    f = jnp.maximum(_dot_t(h, W2_ref[...]) + b2_ref[...][None, :], 0.0)
    out_ref[0] = _dot_t(f, f)


def _logits(x, cond_feats, W1, b1, W2, b2):
    return pl.pallas_call(
        _fused_body,
        grid=(B,),
        in_specs=[
            pl.BlockSpec((1, N, DX), lambda b: (b, 0, 0)),
            pl.BlockSpec((B, COND), lambda b: (0, 0)),
            pl.BlockSpec((HID, DX + COND), lambda b: (0, 0)),
            pl.BlockSpec((HID,), lambda b: (0,)),
            pl.BlockSpec((HID, HID), lambda b: (0, 0)),
            pl.BlockSpec((HID,), lambda b: (0,)),
        ],
        out_specs=pl.BlockSpec((1, N, N), lambda b: (b, 0, 0)),
        out_shape=jax.ShapeDtypeStruct((B, N, N), jnp.float32),
    )(x, cond_feats, W1, b1, W2, b2)


L = 16
NW = 32
ROWS = B * N
RPW = ROWS // NW
R = 8
NBLK = RPW // R
NCH = N // L
CAP = N + L
_NINF = float("-inf")


def _merge_desc(ak, av, bk, bv):
    rbk = lax.rev(bk, (0,))
    rbv = lax.rev(bv, (0,))
    m = ak >= rbk
    tk = jnp.where(m, ak, rbk)
    tv = jnp.where(m, av, rbv)
    uk = jnp.where(m, rbk, ak)
    uv = jnp.where(m, rbv, av)
    tk, tv = plsc.sort_key_val(tk, tv, descending=True)
    uk, uv = plsc.sort_key_val(uk, uv, descending=True)
    return tk, tv, uk, uv


def _sorted_chunk(cand_v, cand_i, j):
    ck, cv = plsc.sort_key_val(cand_v[pl.ds(j * L, L)],
                               cand_i[pl.ds(j * L, L)], descending=True)
    return ck, cv


def _row_topk(buf, cand_v, cand_i, ow, oi, tv, ti, i):
    ninf = jnp.full((L,), _NINF, jnp.float32)

    def p1(j, carry):
        m1a, m2a, m1b, m2b, m1c, m2c, m1d, m2d = carry
        c0 = buf[i, pl.ds((4 * j) * L, L)]
        c1 = buf[i, pl.ds((4 * j + 1) * L, L)]
        c2 = buf[i, pl.ds((4 * j + 2) * L, L)]
        c3 = buf[i, pl.ds((4 * j + 3) * L, L)]
        m2a = jnp.maximum(m2a, jnp.minimum(m1a, c0))
        m1a = jnp.maximum(m1a, c0)
        m2b = jnp.maximum(m2b, jnp.minimum(m1b, c1))
        m1b = jnp.maximum(m1b, c1)
        m2c = jnp.maximum(m2c, jnp.minimum(m1c, c2))
        m1c = jnp.maximum(m1c, c2)
        m2d = jnp.maximum(m2d, jnp.minimum(m1d, c3))
        m1d = jnp.maximum(m1d, c3)
        return m1a, m2a, m1b, m2b, m1c, m2c, m1d, m2d

    m1a, m2a, m1b, m2b, m1c, m2c, m1d, m2d = lax.fori_loop(
        0, NCH // 4, p1, (ninf,) * 8)

    def comb(m1x, m2x, m1y, m2y):
        return (jnp.maximum(m1x, m1y),
                jnp.maximum(jnp.minimum(m1x, m1y), jnp.maximum(m2x, m2y)))

    m1ab, m2ab = comb(m1a, m2a, m1b, m2b)
    m1cd, m2cd = comb(m1c, m2c, m1d, m2d)
    _, m2 = comb(m1ab, m2ab, m1cd, m2cd)
    t0 = jnp.min(m2)

    iota = lax.iota(jnp.int32, L)

    def p2(j, cnt):
        c0 = buf[i, pl.ds((2 * j) * L, L)]
        c1 = buf[i, pl.ds((2 * j + 1) * L, L)]
        msk0 = c0 >= t0
        msk1 = c1 >= t0
        s0 = jnp.sum(msk0.astype(jnp.int32))
        s1 = jnp.sum(msk1.astype(jnp.int32))

        @pl.when(s0 > 0)
        def _():
            plsc.store_compressed(cand_v.at[pl.ds(cnt, L)], c0, mask=msk0)
            plsc.store_compressed(cand_i.at[pl.ds(cnt, L)],
                                  iota + (2 * j) * L, mask=msk0)
        cnt1 = cnt + s0

        @pl.when(s1 > 0)
        def _():
            plsc.store_compressed(cand_v.at[pl.ds(cnt1, L)], c1, mask=msk1)
            plsc.store_compressed(cand_i.at[pl.ds(cnt1, L)],
                                  iota + (2 * j + 1) * L, mask=msk1)
        return cnt1 + s1

    cnt = lax.fori_loop(0, NCH // 2, p2, jnp.int32(0))
    cand_v[pl.ds(cnt, L)] = jnp.full((L,), _NINF, jnp.float32)

    k0, v0 = _sorted_chunk(cand_v, cand_i, 0)
    k1, v1 = _sorted_chunk(cand_v, cand_i, 1)
    hk, hv, lk, lv = _merge_desc(k0, v0, k1, v1)
    nch = (cnt + (L - 1)) // L

    def cond(st):
        return st[0] < nch

    def body(st):
        j, hk, hv, lk, lv = st
        ck, cv = _sorted_chunk(cand_v, cand_i, j)
        yk, yv, _, _ = _merge_desc(lk, lv, ck, cv)
        hk, hv, lk, lv = _merge_desc(hk, hv, yk, yv)
        return j + 1, hk, hv, lk, lv

    _, hk, hv, lk, lv = lax.while_loop(cond, body,
                                       (jnp.int32(2), hk, hv, lk, lv))

    tv[pl.ds(0, L)] = jnp.full((L,), float("inf"), jnp.float32)
    tv[pl.ds(L, L)] = hk
    tv[pl.ds(2 * L, L)] = lk
    tv[pl.ds(3 * L, L)] = jnp.full((L,), _NINF, jnp.float32)
    vp_h = tv[pl.ds(L - 1, L)]
    vn_h = tv[pl.ds(L + 1, L)]
    vp_l = tv[pl.ds(2 * L - 1, L)]
    vn_l = tv[pl.ds(2 * L + 1, L)]
    eqp_h = hk == vp_h
    eqn_h = hk == vn_h
    eqp_l = lk == vp_l
    eqn_l = lk == vn_l
    par = (lax.iota(jnp.int32, L) % 2) == 0
    npar = jnp.logical_not(par)
    ti[pl.ds(0, L)] = lax.iota(jnp.int32, L)
    ti[pl.ds(3 * L, L)] = lax.iota(jnp.int32, L)
    for down in (par, npar, par):
        up = jnp.logical_not(down)
        ti[pl.ds(L, L)] = hv
        ti[pl.ds(2 * L, L)] = lv
        ip_h = ti[pl.ds(L - 1, L)]
        in_h = ti[pl.ds(L + 1, L)]
        ip_l = ti[pl.ds(2 * L - 1, L)]
        in_l = ti[pl.ds(2 * L + 1, L)]
        hv = jnp.where(down & eqn_h & (hv > in_h), in_h,
                       jnp.where(up & eqp_h & (ip_h > hv), ip_h, hv))
        lv = jnp.where(down & eqn_l & (lv > in_l), in_l,
                       jnp.where(up & eqp_l & (ip_l > lv), ip_l, lv))

    mx = jnp.max(hk)
    eh = jnp.exp(hk - mx)
    el = jnp.exp(lk - mx)
    s = jnp.sum(eh) + jnp.sum(el)
    ow[i, pl.ds(0, L)] = eh / s
    ow[i, pl.ds(L, L)] = el / s
    oi[i, pl.ds(0, L)] = hv
    oi[i, pl.ds(L, L)] = lv


def _topk_sc(logits):
    lr = logits.reshape(ROWS, N)
    mesh = plsc.VectorSubcoreMesh(core_axis_name="c", subcore_axis_name="s")

    @functools.partial(
        pl.kernel, mesh=mesh,
        compiler_params=pltpu.CompilerParams(needs_layout_passes=False),
        out_type=[jax.ShapeDtypeStruct((ROWS, K), jnp.float32),
                  jax.ShapeDtypeStruct((ROWS, K), jnp.int32)],
        scratch_types=[
            pltpu.VMEM((R, N), jnp.float32),
            pltpu.VMEM((R, N), jnp.float32),
            pltpu.VMEM((CAP,), jnp.float32),
            pltpu.VMEM((CAP,), jnp.int32),
            pltpu.VMEM((R, K), jnp.float32),
            pltpu.VMEM((R, K), jnp.int32),
            pltpu.VMEM((4 * L,), jnp.float32),
            pltpu.VMEM((4 * L,), jnp.int32),
            pltpu.SemaphoreType.DMA,
            pltpu.SemaphoreType.DMA,
        ],
    )
    def topk_kernel(lg, ow_hbm, oi_hbm, bufa, bufb, cand_v, cand_i,
                    ow, oi, tv, ti, sema, semb):
        wid = lax.axis_index("c") * 16 + lax.axis_index("s")
        base = wid * RPW
        pltpu.async_copy(lg.at[pl.ds(base, R)], bufa, sema)

        def pair(g, carry):
            for t in range(2):
                blk = g * 2 + t
                buf, sem = (bufa, sema) if t == 0 else (bufb, semb)
                obuf, osem = (bufb, semb) if t == 0 else (bufa, sema)
                rbase = base + blk * R
                pltpu.make_async_copy(lg.at[pl.ds(rbase, R)], buf, sem).wait()
                if t == 0:
                    pltpu.async_copy(lg.at[pl.ds(rbase + R, R)], obuf, osem)
                else:
                    @pl.when(blk + 1 < NBLK)
                    def _issue():
                        pltpu.async_copy(lg.at[pl.ds(rbase + R, R)],
                                         obuf, osem)

                def rows(i, c2):
                    _row_topk(buf, cand_v, cand_i, ow, oi, tv, ti, i)
                    return c2
                lax.fori_loop(0, R, rows, 0)
                pltpu.sync_copy(ow, ow_hbm.at[pl.ds(rbase, R)])
                pltpu.sync_copy(oi, oi_hbm.at[pl.ds(rbase, R)])
            return carry

        lax.fori_loop(0, NBLK // 2, pair, 0)

    ow, oi = topk_kernel(lr)
    return ow.reshape(B, N, K), oi.reshape(B, N, K)


def _weight_norm_setup(V, g):
    return g[:, None] * V / jnp.linalg.norm(V, axis=1, keepdims=True)


def kernel(node_feats, coords, cond_feats, V1, g1, b1, V2, g2, b2):
    x = jnp.concatenate([node_feats, coords], axis=-1)
    W1 = _weight_norm_setup(V1, g1)
    W2 = _weight_norm_setup(V2, g2)
    logits = _logits(x, cond_feats, W1, b1, W2, b2)
    edge_weights, topk_idx = _topk_sc(logits)
    return edge_weights, topk_idx


# speedup vs baseline: 4.0303x; 1.0039x over previous
import functools

import jax
import jax.numpy as jnp
from jax import lax
from jax.experimental import pallas as pl
from jax.experimental.pallas import tpu as pltpu
from jax.experimental.pallas import tpu_sc as plsc

B, N = 16, 1024
D_FEAT, D_COORD, COND = 253, 3, 512
DX = D_FEAT + D_COORD
HID = 512
K = 32

_P = lax.Precision.DEFAULT


def _dot_t(a, b, precision=_P):
    return lax.dot_general(a, b, (((1,), (1,)), ((), ())),
                           preferred_element_type=jnp.float32,
                           precision=precision)


def _fused_body(x_ref, cond_ref, W1_ref, b1_ref, W2_ref, b2_ref, out_ref):
    b = pl.program_id(0)
    x = x_ref[0]
    cond = cond_ref[pl.ds(b, 1), :]
    q = jnp.concatenate(
        [x, jnp.broadcast_to(cond, (N, COND))], axis=1)
    h = jnp.maximum(_dot_t(q, W1_ref[...]) + b1_ref[...][None, :], 0.0)
    f = jnp.maximum(_dot_t(h, W2_ref[...]) + b2_ref[...][None, :], 0.0)
    out_ref[0] = _dot_t(f, f)


def _logits(x, cond_feats, W1, b1, W2, b2):
    return pl.pallas_call(
        _fused_body,
        grid=(B,),
        in_specs=[
            pl.BlockSpec((1, N, DX), lambda b: (b, 0, 0)),
            pl.BlockSpec((B, COND), lambda b: (0, 0)),
            pl.BlockSpec((HID, DX + COND), lambda b: (0, 0)),
            pl.BlockSpec((HID,), lambda b: (0,)),
            pl.BlockSpec((HID, HID), lambda b: (0, 0)),
            pl.BlockSpec((HID,), lambda b: (0,)),
        ],
        out_specs=pl.BlockSpec((1, N, N), lambda b: (b, 0, 0)),
        out_shape=jax.ShapeDtypeStruct((B, N, N), jnp.float32),
    )(x, cond_feats, W1, b1, W2, b2)


L = 16
NW = 32
ROWS = B * N
RPW = ROWS // NW
R = 8
NBLK = RPW // R
NCH = N // L
CAP = N + L
_NINF = float("-inf")


def _merge_desc(ak, av, bk, bv):
    rbk = lax.rev(bk, (0,))
    rbv = lax.rev(bv, (0,))
    m = ak >= rbk
    tk = jnp.where(m, ak, rbk)
    tv = jnp.where(m, av, rbv)
    uk = jnp.where(m, rbk, ak)
    uv = jnp.where(m, rbv, av)
    tk, tv = plsc.sort_key_val(tk, tv, descending=True)
    uk, uv = plsc.sort_key_val(uk, uv, descending=True)
    return tk, tv, uk, uv


def _sorted_chunk(cand_v, cand_i, j):
    ck, cv = plsc.sort_key_val(cand_v[pl.ds(j * L, L)],
                               cand_i[pl.ds(j * L, L)], descending=True)
    return ck, cv


def _row_topk(buf, cand_v, cand_i, ow, oi, tv, ti, i):
    ninf = jnp.full((L,), _NINF, jnp.float32)

    def p1(j, carry):
        m1a, m2a, m1b, m2b, m1c, m2c, m1d, m2d = carry
        c0 = buf[i, pl.ds((4 * j) * L, L)]
        c1 = buf[i, pl.ds((4 * j + 1) * L, L)]
        c2 = buf[i, pl.ds((4 * j + 2) * L, L)]
        c3 = buf[i, pl.ds((4 * j + 3) * L, L)]
        m2a = jnp.maximum(m2a, jnp.minimum(m1a, c0))
        m1a = jnp.maximum(m1a, c0)
        m2b = jnp.maximum(m2b, jnp.minimum(m1b, c1))
        m1b = jnp.maximum(m1b, c1)
        m2c = jnp.maximum(m2c, jnp.minimum(m1c, c2))
        m1c = jnp.maximum(m1c, c2)
        m2d = jnp.maximum(m2d, jnp.minimum(m1d, c3))
        m1d = jnp.maximum(m1d, c3)
        return m1a, m2a, m1b, m2b, m1c, m2c, m1d, m2d

    m1a, m2a, m1b, m2b, m1c, m2c, m1d, m2d = lax.fori_loop(
        0, NCH // 4, p1, (ninf,) * 8)

    def comb(m1x, m2x, m1y, m2y):
        return (jnp.maximum(m1x, m1y),
                jnp.maximum(jnp.minimum(m1x, m1y), jnp.maximum(m2x, m2y)))

    m1ab, m2ab = comb(m1a, m2a, m1b, m2b)
    m1cd, m2cd = comb(m1c, m2c, m1d, m2d)
    _, m2 = comb(m1ab, m2ab, m1cd, m2cd)
    t0 = jnp.min(m2)

    iota = lax.iota(jnp.int32, L)

    def one_chunk(j, cntv, c):
        msk = c >= t0
        mi = msk.astype(jnp.int32)
        dst = cntv + plsc.cumsum(mi) - mi
        plsc.store_scatter(cand_v, [dst], c, mask=msk)
        plsc.store_scatter(cand_i, [dst], iota + j * L, mask=msk)
        return cntv + plsc.all_reduce_population_count(msk)

    def p2(j, cntv):
        cntv = one_chunk(4 * j, cntv, buf[i, pl.ds((4 * j) * L, L)])
        cntv = one_chunk(4 * j + 1, cntv, buf[i, pl.ds((4 * j + 1) * L, L)])
        cntv = one_chunk(4 * j + 2, cntv, buf[i, pl.ds((4 * j + 2) * L, L)])
        cntv = one_chunk(4 * j + 3, cntv, buf[i, pl.ds((4 * j + 3) * L, L)])
        return cntv

    cntv = lax.fori_loop(0, NCH // 4, p2, jnp.zeros((L,), jnp.int32))
    cnt = jnp.sum(cntv) // L
    cand_v[pl.ds(cnt, L)] = jnp.full((L,), _NINF, jnp.float32)

    k0, v0 = _sorted_chunk(cand_v, cand_i, 0)
    k1, v1 = _sorted_chunk(cand_v, cand_i, 1)
    hk, hv, lk, lv = _merge_desc(k0, v0, k1, v1)
    nch = (cnt + (L - 1)) // L

    def cond(st):
        return st[0] < nch

    def body(st):
        j, hk, hv, lk, lv = st
        ck, cv = _sorted_chunk(cand_v, cand_i, j)
        yk, yv, _, _ = _merge_desc(lk, lv, ck, cv)
        hk, hv, lk, lv = _merge_desc(hk, hv, yk, yv)
        return j + 1, hk, hv, lk, lv

    _, hk, hv, lk, lv = lax.while_loop(cond, body,
                                       (jnp.int32(2), hk, hv, lk, lv))

    tv[pl.ds(0, L)] = jnp.full((L,), float("inf"), jnp.float32)
    tv[pl.ds(L, L)] = hk
    tv[pl.ds(2 * L, L)] = lk
    tv[pl.ds(3 * L, L)] = jnp.full((L,), _NINF, jnp.float32)
    vp_h = tv[pl.ds(L - 1, L)]
    vn_h = tv[pl.ds(L + 1, L)]
    vp_l = tv[pl.ds(2 * L - 1, L)]
    vn_l = tv[pl.ds(2 * L + 1, L)]
    eqp_h = hk == vp_h
    eqn_h = hk == vn_h
    eqp_l = lk == vp_l
    eqn_l = lk == vn_l
    par = (lax.iota(jnp.int32, L) % 2) == 0
    npar = jnp.logical_not(par)
    ti[pl.ds(0, L)] = lax.iota(jnp.int32, L)
    ti[pl.ds(3 * L, L)] = lax.iota(jnp.int32, L)
    for down in (par, npar, par):
        up = jnp.logical_not(down)
        ti[pl.ds(L, L)] = hv
        ti[pl.ds(2 * L, L)] = lv
        ip_h = ti[pl.ds(L - 1, L)]
        in_h = ti[pl.ds(L + 1, L)]
        ip_l = ti[pl.ds(2 * L - 1, L)]
        in_l = ti[pl.ds(2 * L + 1, L)]
        hv = jnp.where(down & eqn_h & (hv > in_h), in_h,
                       jnp.where(up & eqp_h & (ip_h > hv), ip_h, hv))
        lv = jnp.where(down & eqn_l & (lv > in_l), in_l,
                       jnp.where(up & eqp_l & (ip_l > lv), ip_l, lv))

    mx = jnp.max(hk)
    eh = jnp.exp(hk - mx)
    el = jnp.exp(lk - mx)
    s = jnp.sum(eh) + jnp.sum(el)
    ow[i, pl.ds(0, L)] = eh / s
    ow[i, pl.ds(L, L)] = el / s
    oi[i, pl.ds(0, L)] = hv
    oi[i, pl.ds(L, L)] = lv


def _topk_sc(logits):
    lr = logits.reshape(ROWS, N)
    mesh = plsc.VectorSubcoreMesh(core_axis_name="c", subcore_axis_name="s")

    @functools.partial(
        pl.kernel, mesh=mesh,
        compiler_params=pltpu.CompilerParams(needs_layout_passes=False),
        out_type=[jax.ShapeDtypeStruct((ROWS, K), jnp.float32),
                  jax.ShapeDtypeStruct((ROWS, K), jnp.int32)],
        scratch_types=[
            pltpu.VMEM((R, N), jnp.float32),
            pltpu.VMEM((R, N), jnp.float32),
            pltpu.VMEM((CAP,), jnp.float32),
            pltpu.VMEM((CAP,), jnp.int32),
            pltpu.VMEM((R, K), jnp.float32),
            pltpu.VMEM((R, K), jnp.int32),
            pltpu.VMEM((4 * L,), jnp.float32),
            pltpu.VMEM((4 * L,), jnp.int32),
            pltpu.SemaphoreType.DMA,
            pltpu.SemaphoreType.DMA,
        ],
    )
    def topk_kernel(lg, ow_hbm, oi_hbm, bufa, bufb, cand_v, cand_i,
                    ow, oi, tv, ti, sema, semb):
        wid = lax.axis_index("c") * 16 + lax.axis_index("s")
        base = wid * RPW
        pltpu.async_copy(lg.at[pl.ds(base, R)], bufa, sema)

        def pair(g, carry):
            for t in range(2):
                blk = g * 2 + t
                buf, sem = (bufa, sema) if t == 0 else (bufb, semb)
                obuf, osem = (bufb, semb) if t == 0 else (bufa, sema)
                rbase = base + blk * R
                pltpu.make_async_copy(lg.at[pl.ds(rbase, R)], buf, sem).wait()
                if t == 0:
                    pltpu.async_copy(lg.at[pl.ds(rbase + R, R)], obuf, osem)
                else:
                    @pl.when(blk + 1 < NBLK)
                    def _issue():
                        pltpu.async_copy(lg.at[pl.ds(rbase + R, R)],
                                         obuf, osem)

                def rows(i, c2):
                    _row_topk(buf, cand_v, cand_i, ow, oi, tv, ti, i)
                    return c2
                lax.fori_loop(0, R, rows, 0)
                pltpu.sync_copy(ow, ow_hbm.at[pl.ds(rbase, R)])
                pltpu.sync_copy(oi, oi_hbm.at[pl.ds(rbase, R)])
            return carry

        lax.fori_loop(0, NBLK // 2, pair, 0)

    ow, oi = topk_kernel(lr)
    return ow.reshape(B, N, K), oi.reshape(B, N, K)


def _weight_norm_setup(V, g):
    return g[:, None] * V / jnp.linalg.norm(V, axis=1, keepdims=True)


def kernel(node_feats, coords, cond_feats, V1, g1, b1, V2, g2, b2):
    x = jnp.concatenate([node_feats, coords], axis=-1)
    W1 = _weight_norm_setup(V1, g1)
    W2 = _weight_norm_setup(V2, g2)
    logits = _logits(x, cond_feats, W1, b1, W2, b2)
    edge_weights, topk_idx = _topk_sc(logits)
    return edge_weights, topk_idx


# speedup vs baseline: 4.1728x; 1.0354x over previous
import functools

import jax
import jax.numpy as jnp
from jax import lax
from jax.experimental import pallas as pl
from jax.experimental.pallas import tpu as pltpu
from jax.experimental.pallas import tpu_sc as plsc

B, N = 16, 1024
D_FEAT, D_COORD, COND = 253, 3, 512
DX = D_FEAT + D_COORD
HID = 512
K = 32

_P = lax.Precision.DEFAULT


def _dot_t(a, b, precision=_P):
    return lax.dot_general(a, b, (((1,), (1,)), ((), ())),
                           preferred_element_type=jnp.float32,
                           precision=precision)


def _fused_body(x_ref, cond_ref, W1_ref, b1_ref, W2_ref, b2_ref, out_ref):
    b = pl.program_id(0)
    x = x_ref[0]
    cond = cond_ref[pl.ds(b, 1), :]
    q = jnp.concatenate(
        [x, jnp.broadcast_to(cond, (N, COND))], axis=1)
    h = jnp.maximum(_dot_t(q, W1_ref[...]) + b1_ref[...][None, :], 0.0)
    f = jnp.maximum(_dot_t(h, W2_ref[...]) + b2_ref[...][None, :], 0.0)
    out_ref[0] = _dot_t(f, f)


def _logits(x, cond_feats, W1, b1, W2, b2):
    nb = x.shape[0]
    return pl.pallas_call(
        _fused_body,
        grid=(nb,),
        in_specs=[
            pl.BlockSpec((1, N, DX), lambda b: (b, 0, 0)),
            pl.BlockSpec((nb, COND), lambda b: (0, 0)),
            pl.BlockSpec((HID, DX + COND), lambda b: (0, 0)),
            pl.BlockSpec((HID,), lambda b: (0,)),
            pl.BlockSpec((HID, HID), lambda b: (0, 0)),
            pl.BlockSpec((HID,), lambda b: (0,)),
        ],
        out_specs=pl.BlockSpec((1, N, N), lambda b: (b, 0, 0)),
        out_shape=jax.ShapeDtypeStruct((nb, N, N), jnp.float32),
    )(x, cond_feats, W1, b1, W2, b2)


L = 16
NW = 32
ROWS = B * N
RPW = ROWS // NW
R = 8
NBLK = RPW // R
NCH = N // L
CAP = N + L
_NINF = float("-inf")


def _merge_desc(ak, av, bk, bv):
    rbk = lax.rev(bk, (0,))
    rbv = lax.rev(bv, (0,))
    m = ak >= rbk
    tk = jnp.where(m, ak, rbk)
    tv = jnp.where(m, av, rbv)
    uk = jnp.where(m, rbk, ak)
    uv = jnp.where(m, rbv, av)
    tk, tv = plsc.sort_key_val(tk, tv, descending=True)
    uk, uv = plsc.sort_key_val(uk, uv, descending=True)
    return tk, tv, uk, uv


def _sorted_chunk(cand_v, cand_i, j):
    ck, cv = plsc.sort_key_val(cand_v[pl.ds(j * L, L)],
                               cand_i[pl.ds(j * L, L)], descending=True)
    return ck, cv


def _row_topk(buf, cand_v, cand_i, ow, oi, tv, ti, i):
    ninf = jnp.full((L,), _NINF, jnp.float32)

    def p1(j, carry):
        m1a, m2a, m1b, m2b, m1c, m2c, m1d, m2d = carry
        c0 = buf[i, pl.ds((4 * j) * L, L)]
        c1 = buf[i, pl.ds((4 * j + 1) * L, L)]
        c2 = buf[i, pl.ds((4 * j + 2) * L, L)]
        c3 = buf[i, pl.ds((4 * j + 3) * L, L)]
        m2a = jnp.maximum(m2a, jnp.minimum(m1a, c0))
        m1a = jnp.maximum(m1a, c0)
        m2b = jnp.maximum(m2b, jnp.minimum(m1b, c1))
        m1b = jnp.maximum(m1b, c1)
        m2c = jnp.maximum(m2c, jnp.minimum(m1c, c2))
        m1c = jnp.maximum(m1c, c2)
        m2d = jnp.maximum(m2d, jnp.minimum(m1d, c3))
        m1d = jnp.maximum(m1d, c3)
        return m1a, m2a, m1b, m2b, m1c, m2c, m1d, m2d

    m1a, m2a, m1b, m2b, m1c, m2c, m1d, m2d = lax.fori_loop(
        0, NCH // 4, p1, (ninf,) * 8)

    def comb(m1x, m2x, m1y, m2y):
        return (jnp.maximum(m1x, m1y),
                jnp.maximum(jnp.minimum(m1x, m1y), jnp.maximum(m2x, m2y)))

    m1ab, m2ab = comb(m1a, m2a, m1b, m2b)
    m1cd, m2cd = comb(m1c, m2c, m1d, m2d)
    _, m2 = comb(m1ab, m2ab, m1cd, m2cd)
    t0 = jnp.min(m2)

    iota = lax.iota(jnp.int32, L)

    def one_chunk(j, cntv, c):
        msk = c >= t0
        mi = msk.astype(jnp.int32)
        dst = cntv + plsc.cumsum(mi) - mi
        plsc.store_scatter(cand_v, [dst], c, mask=msk)
        plsc.store_scatter(cand_i, [dst], iota + j * L, mask=msk)
        return cntv + plsc.all_reduce_population_count(msk)

    def p2(j, cntv):
        cntv = one_chunk(4 * j, cntv, buf[i, pl.ds((4 * j) * L, L)])
        cntv = one_chunk(4 * j + 1, cntv, buf[i, pl.ds((4 * j + 1) * L, L)])
        cntv = one_chunk(4 * j + 2, cntv, buf[i, pl.ds((4 * j + 2) * L, L)])
        cntv = one_chunk(4 * j + 3, cntv, buf[i, pl.ds((4 * j + 3) * L, L)])
        return cntv

    cntv = lax.fori_loop(0, NCH // 4, p2, jnp.zeros((L,), jnp.int32))
    cnt = jnp.sum(cntv) // L
    cand_v[pl.ds(cnt, L)] = jnp.full((L,), _NINF, jnp.float32)

    k0, v0 = _sorted_chunk(cand_v, cand_i, 0)
    k1, v1 = _sorted_chunk(cand_v, cand_i, 1)
    hk, hv, lk, lv = _merge_desc(k0, v0, k1, v1)
    nch = (cnt + (L - 1)) // L

    def cond(st):
        return st[0] < nch

    def body(st):
        j, hk, hv, lk, lv = st
        ck, cv = _sorted_chunk(cand_v, cand_i, j)
        yk, yv, _, _ = _merge_desc(lk, lv, ck, cv)
        hk, hv, lk, lv = _merge_desc(hk, hv, yk, yv)
        return j + 1, hk, hv, lk, lv

    _, hk, hv, lk, lv = lax.while_loop(cond, body,
                                       (jnp.int32(2), hk, hv, lk, lv))

    tv[pl.ds(0, L)] = jnp.full((L,), float("inf"), jnp.float32)
    tv[pl.ds(L, L)] = hk
    tv[pl.ds(2 * L, L)] = lk
    tv[pl.ds(3 * L, L)] = jnp.full((L,), _NINF, jnp.float32)
    vp_h = tv[pl.ds(L - 1, L)]
    vn_h = tv[pl.ds(L + 1, L)]
    vp_l = tv[pl.ds(2 * L - 1, L)]
    vn_l = tv[pl.ds(2 * L + 1, L)]
    eqp_h = hk == vp_h
    eqn_h = hk == vn_h
    eqp_l = lk == vp_l
    eqn_l = lk == vn_l
    par = (lax.iota(jnp.int32, L) % 2) == 0
    npar = jnp.logical_not(par)
    ti[pl.ds(0, L)] = lax.iota(jnp.int32, L)
    ti[pl.ds(3 * L, L)] = lax.iota(jnp.int32, L)
    for down in (par, npar, par):
        up = jnp.logical_not(down)
        ti[pl.ds(L, L)] = hv
        ti[pl.ds(2 * L, L)] = lv
        ip_h = ti[pl.ds(L - 1, L)]
        in_h = ti[pl.ds(L + 1, L)]
        ip_l = ti[pl.ds(2 * L - 1, L)]
        in_l = ti[pl.ds(2 * L + 1, L)]
        hv = jnp.where(down & eqn_h & (hv > in_h), in_h,
                       jnp.where(up & eqp_h & (ip_h > hv), ip_h, hv))
        lv = jnp.where(down & eqn_l & (lv > in_l), in_l,
                       jnp.where(up & eqp_l & (ip_l > lv), ip_l, lv))

    mx = jnp.max(hk)
    eh = jnp.exp(hk - mx)
    el = jnp.exp(lk - mx)
    s = jnp.sum(eh) + jnp.sum(el)
    ow[i, pl.ds(0, L)] = eh / s
    ow[i, pl.ds(L, L)] = el / s
    oi[i, pl.ds(0, L)] = hv
    oi[i, pl.ds(L, L)] = lv


def _topk_sc(logits):
    nb = logits.shape[0]
    rows = nb * N
    rpw = rows // NW
    nblk = rpw // R
    lr = logits.reshape(rows, N)
    mesh = plsc.VectorSubcoreMesh(core_axis_name="c", subcore_axis_name="s")

    @functools.partial(
        pl.kernel, mesh=mesh,
        compiler_params=pltpu.CompilerParams(needs_layout_passes=False),
        out_type=[jax.ShapeDtypeStruct((rows, K), jnp.float32),
                  jax.ShapeDtypeStruct((rows, K), jnp.int32)],
        scratch_types=[
            pltpu.VMEM((R, N), jnp.float32),
            pltpu.VMEM((R, N), jnp.float32),
            pltpu.VMEM((CAP,), jnp.float32),
            pltpu.VMEM((CAP,), jnp.int32),
            pltpu.VMEM((R, K), jnp.float32),
            pltpu.VMEM((R, K), jnp.int32),
            pltpu.VMEM((4 * L,), jnp.float32),
            pltpu.VMEM((4 * L,), jnp.int32),
            pltpu.SemaphoreType.DMA,
            pltpu.SemaphoreType.DMA,
        ],
    )
    def topk_kernel(lg, ow_hbm, oi_hbm, bufa, bufb, cand_v, cand_i,
                    ow, oi, tv, ti, sema, semb):
        wid = lax.axis_index("c") * 16 + lax.axis_index("s")
        base = wid * rpw
        pltpu.async_copy(lg.at[pl.ds(base, R)], bufa, sema)

        def pair(g, carry):
            for t in range(2):
                blk = g * 2 + t
                buf, sem = (bufa, sema) if t == 0 else (bufb, semb)
                obuf, osem = (bufb, semb) if t == 0 else (bufa, sema)
                rbase = base + blk * R
                pltpu.make_async_copy(lg.at[pl.ds(rbase, R)], buf, sem).wait()
                if t == 0:
                    pltpu.async_copy(lg.at[pl.ds(rbase + R, R)], obuf, osem)
                else:
                    @pl.when(blk + 1 < nblk)
                    def _issue():
                        pltpu.async_copy(lg.at[pl.ds(rbase + R, R)],
                                         obuf, osem)

                def rows(i, c2):
                    _row_topk(buf, cand_v, cand_i, ow, oi, tv, ti, i)
                    return c2
                lax.fori_loop(0, R, rows, 0)
                pltpu.sync_copy(ow, ow_hbm.at[pl.ds(rbase, R)])
                pltpu.sync_copy(oi, oi_hbm.at[pl.ds(rbase, R)])
            return carry

        lax.fori_loop(0, nblk // 2, pair, 0)

    ow, oi = topk_kernel(lr)
    return ow.reshape(nb, N, K), oi.reshape(nb, N, K)


def _weight_norm_setup(V, g):
    return g[:, None] * V / jnp.linalg.norm(V, axis=1, keepdims=True)


def kernel(node_feats, coords, cond_feats, V1, g1, b1, V2, g2, b2):
    x = jnp.concatenate([node_feats, coords], axis=-1)
    W1 = _weight_norm_setup(V1, g1)
    W2 = _weight_norm_setup(V2, g2)
    hb = B // 2
    l0 = _logits(x[:hb], cond_feats[:hb], W1, b1, W2, b2)
    l1 = _logits(x[hb:], cond_feats[hb:], W1, b1, W2, b2)
    ew0, ti0 = _topk_sc(l0)
    ew1, ti1 = _topk_sc(l1)
    edge_weights = jnp.concatenate([ew0, ew1], axis=0)
    topk_idx = jnp.concatenate([ti0, ti1], axis=0)
    return edge_weights, topk_idx
